# baseline (speedup 1.0000x reference)
import functools
import math

import jax
import jax.numpy as jnp
from jax import lax
from jax.experimental import pallas as pl
from jax.experimental.pallas import tpu as pltpu
from jax.experimental.pallas import tpu_sc as plsc

_NUM_CORES = 2
_NUM_SUBCORES = 16
_NUM_WORKERS = _NUM_CORES * _NUM_SUBCORES
_LANES = 16
_NBUF = 4
_PRIME = 3


def _make_sc_gather(batch: int, seq: int, vocab: int, d_model: int):
  n_tokens = batch * seq
  assert n_tokens % _NUM_WORKERS == 0
  per_worker = n_tokens // _NUM_WORKERS
  assert seq % per_worker == 0
  workers_per_seq = seq // per_worker
  chunk = 16
  while per_worker % (chunk * _NBUF):
    chunk //= 2
  assert chunk > 0
  n_chunks = per_worker // chunk
  n_groups = n_chunks // _NBUF
  vecs_per_row = d_model // _LANES
  scale = jnp.float32(math.sqrt(d_model))
  last_start = n_chunks - _PRIME
  slack = _NBUF - _PRIME

  mesh = plsc.VectorSubcoreMesh(core_axis_name="c", subcore_axis_name="s")

  @functools.partial(
      pl.kernel,
      mesh=mesh,
      out_type=jax.ShapeDtypeStruct((batch, seq, d_model), jnp.float32),
      scratch_types=[
          pltpu.VMEM((per_worker,), jnp.int32),
          *([pltpu.VMEM((chunk, d_model), jnp.float32)] * _NBUF),
          *([pltpu.SemaphoreType.DMA] * (2 * _NBUF)),
      ],
  )
  def gather_kernel(tok_hbm, tab_hbm, out_hbm, idx_v, *bufs_and_sems):
    bufs = bufs_and_sems[:_NBUF]
    gsem = bufs_and_sems[_NBUF:2 * _NBUF]
    ssem = bufs_and_sems[2 * _NBUF:]

    wid = lax.axis_index("s") * _NUM_CORES + lax.axis_index("c")
    row = wid // workers_per_seq
    col0 = (wid % workers_per_seq) * per_worker

    pltpu.sync_copy(tok_hbm.at[row, pl.ds(col0, per_worker)], idx_v)

    def gather_cp(g, b):
      return pltpu.make_async_copy(
          tab_hbm.at[idx_v.at[pl.ds(g * chunk, chunk)]], bufs[b], gsem[b])

    def scatter_cp(g, b):
      return pltpu.make_async_copy(
          bufs[b], out_hbm.at[row, pl.ds(col0 + g * chunk, chunk), :],
          ssem[b])

    for j in range(_PRIME):
      gather_cp(j, j).start()

    @pl.loop(0, n_groups)
    def _(grp):
      for b in range(_NBUF):
        g = grp * _NBUF + b
        nb = (b + _PRIME) % _NBUF

        @pl.when(jnp.logical_and(g >= slack, g < last_start))
        def _():
          scatter_cp(g - slack, nb).wait()

        @pl.when(g < last_start)
        def _():
          gather_cp(g + _PRIME, nb).start()

        gather_cp(g, b).wait()
        cur = bufs[b]

        @plsc.parallel_loop(0, chunk * vecs_per_row, unroll=8)
        def _(i):
          r = i // vecs_per_row
          sl = pl.ds((i % vecs_per_row) * _LANES, _LANES)
          cur[r, sl] = cur[r, sl] * scale

        scatter_cp(g, b).start()

    for g in range(n_chunks - _NBUF, n_chunks):
      scatter_cp(g, g % _NBUF).wait()

  return gather_kernel


def kernel(token, table):
  vocab, d_model = table.shape
  batch, seq = token.shape
  tok = token.astype(jnp.int32)
  return _make_sc_gather(batch, seq, vocab, d_model)(tok, table)


# speedup vs baseline: 1.0181x; 1.0181x over previous
import functools
import math

import jax
import jax.numpy as jnp
from jax import lax
from jax.experimental import pallas as pl
from jax.experimental.pallas import tpu as pltpu
from jax.experimental.pallas import tpu_sc as plsc

_NUM_CORES = 2
_NUM_SUBCORES = 16
_NUM_WORKERS = _NUM_CORES * _NUM_SUBCORES
_LANES = 16
_NBUF = 8
_PRIME = 7


def _make_sc_gather(batch: int, seq: int, vocab: int, d_model: int):
  n_tokens = batch * seq
  assert n_tokens % _NUM_WORKERS == 0
  per_worker = n_tokens // _NUM_WORKERS
  assert seq % per_worker == 0
  workers_per_seq = seq // per_worker
  chunk = 8
  while per_worker % (chunk * _NBUF):
    chunk //= 2
  assert chunk > 0
  n_chunks = per_worker // chunk
  n_groups = n_chunks // _NBUF
  vecs_per_row = d_model // _LANES
  scale = jnp.float32(math.sqrt(d_model))
  last_start = n_chunks - _PRIME
  slack = _NBUF - _PRIME

  mesh = plsc.VectorSubcoreMesh(core_axis_name="c", subcore_axis_name="s")

  @functools.partial(
      pl.kernel,
      mesh=mesh,
      out_type=jax.ShapeDtypeStruct((batch, seq, d_model), jnp.float32),
      scratch_types=[
          pltpu.VMEM((per_worker,), jnp.int32),
          *([pltpu.VMEM((chunk, d_model), jnp.float32)] * _NBUF),
          *([pltpu.SemaphoreType.DMA] * (2 * _NBUF)),
      ],
  )
  def gather_kernel(tok_hbm, tab_hbm, out_hbm, idx_v, *bufs_and_sems):
    bufs = bufs_and_sems[:_NBUF]
    gsem = bufs_and_sems[_NBUF:2 * _NBUF]
    ssem = bufs_and_sems[2 * _NBUF:]

    wid = lax.axis_index("s") * _NUM_CORES + lax.axis_index("c")
    row = wid // workers_per_seq
    col0 = (wid % workers_per_seq) * per_worker

    pltpu.sync_copy(tok_hbm.at[row, pl.ds(col0, per_worker)], idx_v)

    def gather_cp(g, b):
      return pltpu.make_async_copy(
          tab_hbm.at[idx_v.at[pl.ds(g * chunk, chunk)]], bufs[b], gsem[b])

    def scatter_cp(g, b):
      return pltpu.make_async_copy(
          bufs[b], out_hbm.at[row, pl.ds(col0 + g * chunk, chunk), :],
          ssem[b])

    for j in range(_PRIME):
      gather_cp(j, j).start()

    @pl.loop(0, n_groups)
    def _(grp):
      for b in range(_NBUF):
        g = grp * _NBUF + b
        nb = (b + _PRIME) % _NBUF

        @pl.when(jnp.logical_and(g >= slack, g < last_start))
        def _():
          scatter_cp(g - slack, nb).wait()

        @pl.when(g < last_start)
        def _():
          gather_cp(g + _PRIME, nb).start()

        gather_cp(g, b).wait()
        cur = bufs[b]

        @plsc.parallel_loop(0, chunk * vecs_per_row, unroll=8)
        def _(i):
          r = i // vecs_per_row
          sl = pl.ds((i % vecs_per_row) * _LANES, _LANES)
          cur[r, sl] = cur[r, sl] * scale

        scatter_cp(g, b).start()

    for g in range(n_chunks - _NBUF, n_chunks):
      scatter_cp(g, g % _NBUF).wait()

  return gather_kernel


def kernel(token, table):
  vocab, d_model = table.shape
  batch, seq = token.shape
  tok = token.astype(jnp.int32)
  return _make_sc_gather(batch, seq, vocab, d_model)(tok, table)
